# baseline (speedup 1.0000x reference)
import functools

import jax
import jax.numpy as jnp
from jax import lax
from jax.experimental import pallas as pl
from jax.experimental.pallas import tpu as pltpu
from jax.experimental.pallas import tpu_sc as plsc

_LANES = 16


def _embed_sum_body(ctx, embed, idx_hbm, emb_hbm, out_hbm,
                    idx_v, rows_v, acc_v, sem0, sem1):
    nch = embed // _LANES
    half = (ctx // 2 + 7) // 8 * 8

    pltpu.sync_copy(idx_hbm, idx_v)
    cp0 = pltpu.async_copy(emb_hbm.at[idx_v.at[pl.ds(0, half)]],
                           rows_v.at[pl.ds(0, half)], sem0)
    cp1 = pltpu.async_copy(emb_hbm.at[idx_v.at[pl.ds(half, ctx - half)]],
                           rows_v.at[pl.ds(half, ctx - half)], sem1)

    def row_sum4(j, acc):
        r = j * 4
        return tuple(acc[ch]
                     + (rows_v[r, pl.ds(ch * _LANES, _LANES)]
                        + rows_v[r + 1, pl.ds(ch * _LANES, _LANES)])
                     + (rows_v[r + 2, pl.ds(ch * _LANES, _LANES)]
                        + rows_v[r + 3, pl.ds(ch * _LANES, _LANES)])
                     for ch in range(nch))

    cp0.wait()
    acc = lax.fori_loop(0, half // 4, row_sum4,
                        tuple(jnp.zeros((_LANES,), jnp.float32)
                              for _ in range(nch)))
    cp1.wait()
    acc = lax.fori_loop(half // 4, ctx // 4, row_sum4, acc)
    for ch in range(nch):
        acc_v[pl.ds(ch * _LANES, _LANES)] = acc[ch]
    pltpu.sync_copy(acc_v, out_hbm.at[0])


def _embed_sum_sc(idx, emb):
    embed = emb.shape[1]
    ctx = idx.shape[0]
    mesh = plsc.VectorSubcoreMesh(
        core_axis_name="c", subcore_axis_name="s",
        num_cores=1, num_subcores=1)
    kern = pl.kernel(
        functools.partial(_embed_sum_body, ctx, embed),
        out_type=jax.ShapeDtypeStruct((1, embed), jnp.float32),
        mesh=mesh,
        scratch_types=[
            pltpu.VMEM((ctx,), jnp.int32),
            pltpu.VMEM((ctx, embed), jnp.float32),
            pltpu.VMEM((embed,), jnp.float32),
            pltpu.SemaphoreType.DMA,
            pltpu.SemaphoreType.DMA,
        ],
    )
    return kern(idx, emb)


_TILE = 32768


def _matvec_body(e_ref, w_ref, b_ref, o_ref):
    o_ref[...] = jax.lax.dot_general(
        e_ref[...], w_ref[...],
        dimension_numbers=(((1,), (1,)), ((), ())),
        preferred_element_type=jnp.float32) + b_ref[...].reshape(1, -1)


def _matvec_tc(partials, W, b):
    vocab, embed = W.shape
    grid = (vocab + _TILE - 1) // _TILE
    return pl.pallas_call(
        _matvec_body,
        grid=(grid,),
        in_specs=[
            pl.BlockSpec((1, embed), lambda i: (0, 0)),
            pl.BlockSpec((_TILE, embed), lambda i: (i, 0)),
            pl.BlockSpec((_TILE,), lambda i: (i,)),
        ],
        out_specs=pl.BlockSpec((1, _TILE), lambda i: (0, i)),
        out_shape=jax.ShapeDtypeStruct((1, vocab), jnp.float32),
    )(partials, W, b)


def kernel(inputs, emb, W, b):
    idx = inputs.astype(jnp.int32)
    partials = _embed_sum_sc(idx, emb)
    return _matvec_tc(partials, W, b)


# speedup vs baseline: 1.0631x; 1.0631x over previous
import functools

import jax
import jax.numpy as jnp
from jax import lax
from jax.experimental import pallas as pl
from jax.experimental.pallas import tpu as pltpu
from jax.experimental.pallas import tpu_sc as plsc

_LANES = 16


def _embed_sum_body(ctx, embed, idx_hbm, emb_hbm, out_hbm,
                    idx_v, rows_v, acc_v, sem0, sem1):
    nch = embed // _LANES
    half = (ctx // 2 + 7) // 8 * 8

    pltpu.sync_copy(idx_hbm, idx_v)
    cp0 = pltpu.async_copy(emb_hbm.at[idx_v.at[pl.ds(0, half)]],
                           rows_v.at[pl.ds(0, half)], sem0)
    cp1 = pltpu.async_copy(emb_hbm.at[idx_v.at[pl.ds(half, ctx - half)]],
                           rows_v.at[pl.ds(half, ctx - half)], sem1)

    def row_sum4(j, acc):
        r = j * 4
        return tuple(acc[ch]
                     + (rows_v[r, pl.ds(ch * _LANES, _LANES)]
                        + rows_v[r + 1, pl.ds(ch * _LANES, _LANES)])
                     + (rows_v[r + 2, pl.ds(ch * _LANES, _LANES)]
                        + rows_v[r + 3, pl.ds(ch * _LANES, _LANES)])
                     for ch in range(nch))

    cp0.wait()
    acc = lax.fori_loop(0, half // 4, row_sum4,
                        tuple(jnp.zeros((_LANES,), jnp.float32)
                              for _ in range(nch)))
    cp1.wait()
    acc = lax.fori_loop(half // 4, ctx // 4, row_sum4, acc)
    for ch in range(nch):
        acc_v[pl.ds(ch * _LANES, _LANES)] = acc[ch]
    pltpu.sync_copy(acc_v, out_hbm.at[0])


def _embed_sum_sc(idx, emb):
    embed = emb.shape[1]
    ctx = idx.shape[0]
    mesh = plsc.VectorSubcoreMesh(
        core_axis_name="c", subcore_axis_name="s",
        num_cores=1, num_subcores=1)
    kern = pl.kernel(
        functools.partial(_embed_sum_body, ctx, embed),
        out_type=jax.ShapeDtypeStruct((1, embed), jnp.float32),
        mesh=mesh,
        scratch_types=[
            pltpu.VMEM((ctx,), jnp.int32),
            pltpu.VMEM((ctx, embed), jnp.float32),
            pltpu.VMEM((embed,), jnp.float32),
            pltpu.SemaphoreType.DMA,
            pltpu.SemaphoreType.DMA,
        ],
    )
    return kern(idx, emb)


_TILE = 16384


def _matvec_body(e_ref, w_ref, b_ref, o_ref):
    o_ref[...] = jax.lax.dot_general(
        e_ref[...], w_ref[...],
        dimension_numbers=(((1,), (1,)), ((), ())),
        preferred_element_type=jnp.float32) + b_ref[...].reshape(1, -1)


def _matvec_tc(partials, W, b):
    vocab, embed = W.shape
    grid = (vocab + _TILE - 1) // _TILE
    return pl.pallas_call(
        _matvec_body,
        grid=(grid,),
        in_specs=[
            pl.BlockSpec((1, embed), lambda i: (0, 0)),
            pl.BlockSpec((_TILE, embed), lambda i: (i, 0)),
            pl.BlockSpec((_TILE,), lambda i: (i,)),
        ],
        out_specs=pl.BlockSpec((1, _TILE), lambda i: (0, i)),
        out_shape=jax.ShapeDtypeStruct((1, vocab), jnp.float32),
    )(partials, W, b)


def kernel(inputs, emb, W, b):
    idx = inputs.astype(jnp.int32)
    partials = _embed_sum_sc(idx, emb)
    return _matvec_tc(partials, W, b)
